# baseline (speedup 1.0000x reference)
import functools

import jax
import jax.numpy as jnp
from jax import lax
from jax.experimental import pallas as pl
from jax.experimental.pallas import tpu as pltpu
from jax.experimental.pallas import tpu_sc as plsc

N = 10000
E = 320000
F = 128
NS = 16
NC = 2
BLK = 128
K = 10
G = 16
EPT = G * K * BLK
EPAD = NS * EPT
NPAD = 10240
DT = NPAD // NS
RPT = NPAD // NS
GB = 128
NGB = RPT // GB

_mesh = plsc.VectorSubcoreMesh(core_axis_name="c", subcore_axis_name="s")


@functools.partial(
    pl.kernel,
    out_type=(
        jax.ShapeDtypeStruct((NC, NPAD), jnp.float32),
        jax.ShapeDtypeStruct((NPAD, F), jnp.float32),
    ),
    mesh=_mesh,
    scratch_types=[
        pltpu.VMEM_SHARED((NPAD,), jnp.float32),
        pltpu.VMEM((K, BLK), jnp.int32),
        pltpu.VMEM((BLK,), jnp.float32),
        pltpu.VMEM((NGB, GB), jnp.int32),
        pltpu.VMEM((GB, F), jnp.float32),
        pltpu.SemaphoreType.DMA,
    ],
)
def _prep(sd_hbm, perm_hbm, x_hbm, ones_hbm, zeros_hbm,
          degs_hbm, xg_hbm,
          acc, idx_v, ones_v, perm_v, rows_v, sem):
    c = lax.axis_index("c")
    s = lax.axis_index("s")
    pltpu.sync_copy(zeros_hbm, acc.at[pl.ds(s * DT, DT)])
    pltpu.sync_copy(ones_hbm, ones_v)
    plsc.subcore_barrier()

    def hist_group(g, carry):
        pltpu.sync_copy(sd_hbm.at[c, s, g], idx_v)

        def hist(j, carry2):
            pltpu.sync_copy(ones_v, acc.at[idx_v.at[j]], add=True)
            return carry2

        return lax.fori_loop(0, K, hist, carry)

    lax.fori_loop(0, G, hist_group, 0)
    plsc.subcore_barrier()
    pltpu.sync_copy(acc.at[pl.ds(s * DT, DT)], degs_hbm.at[c, pl.ds(s * DT, DT)])

    @pl.when(c == 0)
    def _gather_perm():
        pltpu.sync_copy(perm_hbm.at[s], perm_v)

        def gblk(k, carry):
            pltpu.async_copy(x_hbm.at[perm_v.at[k]], rows_v, sem).wait()
            pltpu.sync_copy(rows_v, xg_hbm.at[pl.ds(s * RPT + k * GB, GB)])
            return carry

        lax.fori_loop(0, NGB, gblk, 0)


HF = F // 2
NBUF = 5
LC = 632
TAIL = N - (NS - 1) * LC


@functools.partial(
    pl.kernel,
    out_type=jax.ShapeDtypeStruct((NC, 2, NPAD, HF), jnp.float32),
    mesh=_mesh,
    scratch_types=[
        pltpu.VMEM_SHARED((N, HF), jnp.float32),
        pltpu.VMEM_SHARED((NPAD, HF), jnp.float32),
        pltpu.VMEM((2, K, BLK), jnp.int32),
        pltpu.VMEM((2, K, BLK), jnp.int32),
        pltpu.VMEM((NBUF, BLK, HF), jnp.float32),
        pltpu.SemaphoreType.DMA,
        pltpu.SemaphoreType.DMA,
        pltpu.SemaphoreType.DMA,
        pltpu.SemaphoreType.DMA,
        pltpu.SemaphoreType.DMA,
    ],
    compiler_params=pltpu.CompilerParams(use_tc_tiling_on_sc=False),
)
def _spmm(tlo_hbm, thi_hbm, gidx_hbm, dst_hbm, zrows_hbm,
          out_hbm,
          tbl, acc, gi_v, di_v, rows_v, sem0, sem1, sem2, sem3, sem4):
    sems = (sem0, sem1, sem2, sem3, sem4)
    c = lax.axis_index("c")
    s = lax.axis_index("s")

    for p, t_hbm in enumerate((tlo_hbm, thi_hbm)):
        @pl.when(s < NS - 1)
        def _load_main():
            pltpu.sync_copy(t_hbm.at[pl.ds(c * N + s * LC, LC)],
                            tbl.at[pl.ds(s * LC, LC)])

        @pl.when(s == NS - 1)
        def _load_tail():
            pltpu.sync_copy(t_hbm.at[pl.ds(c * N + (NS - 1) * LC, TAIL)],
                            tbl.at[pl.ds((NS - 1) * LC, TAIL)])

        pltpu.sync_copy(zrows_hbm, acc.at[pl.ds(s * RPT, RPT)])
        pltpu.sync_copy(gidx_hbm.at[s, 0], gi_v.at[0])
        pltpu.sync_copy(dst_hbm.at[s, 0], di_v.at[0])
        plsc.subcore_barrier()

        for j0 in range(NBUF - 1):
            pltpu.async_copy(tbl.at[gi_v.at[0, j0]], rows_v.at[j0], sems[j0])

        def group(g, carry):
            cur = lax.rem(g, 2)
            nxt = 1 - cur

            @pl.when(g + 1 < G)
            def _stage_next():
                pltpu.sync_copy(gidx_hbm.at[s, g + 1], gi_v.at[nxt])
                pltpu.sync_copy(dst_hbm.at[s, g + 1], di_v.at[nxt])

            for j in range(K):
                b = j % NBUF
                ja = j + NBUF - 1
                ba = ja % NBUF
                if ja < K:
                    pltpu.async_copy(tbl.at[gi_v.at[cur, ja]],
                                     rows_v.at[ba], sems[ba])
                else:
                    @pl.when(g + 1 < G)
                    def _fire_cross():
                        pltpu.async_copy(tbl.at[gi_v.at[nxt, ja - K]],
                                         rows_v.at[ba], sems[ba])
                pltpu.make_async_copy(
                    tbl.at[gi_v.at[cur, j]], rows_v.at[b], sems[b]).wait()
                pltpu.sync_copy(rows_v.at[b], acc.at[di_v.at[cur, j]], add=True)
            return carry

        lax.fori_loop(0, G, group, 0)
        plsc.subcore_barrier()
        pltpu.sync_copy(acc.at[pl.ds(s * RPT, RPT)],
                        out_hbm.at[c, p, pl.ds(s * RPT, RPT)])
        if p == 0:
            plsc.subcore_barrier()


def _norm(d):
    return jnp.where(d > 0.5, lax.rsqrt(d), 0.0)


def _t1_body(x_ref, xg_ref, ds_ref, lo_ref, hi_ref):
    ns = _norm(ds_ref[...])
    lo_ref[0:N] = x_ref[:, 0:HF] * ns
    lo_ref[N:2 * N] = xg_ref[0:N, 0:HF] * ns
    hi_ref[0:N] = x_ref[:, HF:F] * ns
    hi_ref[N:2 * N] = xg_ref[0:N, HF:F] * ns


def _mm_body(glo_ref, ghi_ref, dd_ref, ds_ref, w_ref, b_ref, lo_ref, hi_ref):
    g = jnp.concatenate([glo_ref[...], ghi_ref[...]], axis=1)
    g = g * _norm(dd_ref[...])
    h = jnp.dot(g, w_ref[...], preferred_element_type=jnp.float32) + b_ref[...]
    h = jnp.maximum(h, 0.0) * _norm(ds_ref[...])
    lo_ref[...] = h[:, 0:HF]
    hi_ref[...] = h[:, HF:F]


def _fin_body(glo_ref, ghi_ref, dd_ref, w2_ref, b2_ref, wd_ref, o_ref):
    gs = jnp.concatenate([glo_ref[...], ghi_ref[...]], axis=1)
    gs = gs * _norm(dd_ref[...])
    gp = gs[0:N]
    gn = gs[N:2 * N]
    u = jnp.sum(gp, axis=0, keepdims=True) * (1.0 / N)
    sm = jnp.dot(u, w2_ref[...], preferred_element_type=jnp.float32) + b2_ref[...]
    tt = (((1,), (1,)), ((), ()))
    vv = lax.dot_general(sm, wd_ref[...], tt,
                         preferred_element_type=jnp.float32)
    w2v = lax.dot_general(vv, w2_ref[...], tt,
                          preferred_element_type=jnp.float32)
    cc = jnp.sum(b2_ref[...] * vv, axis=1, keepdims=True)
    logits = jnp.sum(gs * w2v, axis=1, keepdims=True) + cc
    lp = logits[0:N]
    ln = logits[N:2 * N]

    def softplus(z):
        return jnp.maximum(z, 0.0) + jnp.log1p(jnp.exp(-jnp.abs(z)))

    l1 = jnp.sum(softplus(-lp), axis=0, keepdims=True) * (1.0 / N)
    l2 = jnp.sum(softplus(ln), axis=0, keepdims=True) * (1.0 / N)
    o_ref[...] = l1 + l2


def kernel(features, edge_index, perm, W1, b1, W2, b2, Wd):
    src = edge_index[0]
    dst = edge_index[1]
    npad = EPAD - E
    junk = (N + jax.lax.iota(jnp.int32, npad) % (NPAD - N))
    src_h = jnp.concatenate([src, junk]).reshape(NS, G, K, BLK)
    dst_p = jnp.concatenate([dst, junk]).reshape(NS, G, K, BLK)
    sd = jnp.stack([src_h, dst_p])
    gidx = jnp.concatenate(
        [src, jnp.zeros((npad,), jnp.int32)]).reshape(NS, G, K, BLK)
    perm_pad = jnp.concatenate(
        [perm, jnp.zeros((NPAD - N,), jnp.int32)]).reshape(NS, NGB, GB)
    ones = jnp.ones((BLK,), jnp.float32)
    zeros = jnp.zeros((DT,), jnp.float32)
    zrows = jnp.zeros((RPT, HF), jnp.float32)

    degs, xg = _prep(sd, perm_pad, features, ones, zeros)
    ds_col = degs[0, :N].reshape(N, 1)
    dd_col = degs[1, :N].reshape(N, 1)
    dd2 = jnp.concatenate([dd_col, dd_col], axis=0)
    ds2 = jnp.concatenate([ds_col, ds_col], axis=0)

    half = jax.ShapeDtypeStruct((2 * N, HF), jnp.float32)
    t1_lo, t1_hi = pl.pallas_call(
        _t1_body, out_shape=(half, half))(features, xg, ds_col)

    g1 = _spmm(t1_lo, t1_hi, gidx, dst_p, zrows)
    g1_lo = g1[:, 0, :N].reshape(2 * N, HF)
    g1_hi = g1[:, 1, :N].reshape(2 * N, HF)

    RB = 2000
    t2_lo, t2_hi = pl.pallas_call(
        _mm_body,
        grid=(2 * N // RB,),
        in_specs=[
            pl.BlockSpec((RB, HF), lambda i: (i, 0)),
            pl.BlockSpec((RB, HF), lambda i: (i, 0)),
            pl.BlockSpec((RB, 1), lambda i: (i, 0)),
            pl.BlockSpec((RB, 1), lambda i: (i, 0)),
            pl.BlockSpec((F, F), lambda i: (0, 0)),
            pl.BlockSpec((1, F), lambda i: (0, 0)),
        ],
        out_specs=[
            pl.BlockSpec((RB, HF), lambda i: (i, 0)),
            pl.BlockSpec((RB, HF), lambda i: (i, 0)),
        ],
        out_shape=(half, half))(
            g1_lo, g1_hi, dd2, ds2, W1, b1.reshape(1, F))

    g2 = _spmm(t2_lo, t2_hi, gidx, dst_p, zrows)
    g2_lo = g2[:, 0, :N].reshape(2 * N, HF)
    g2_hi = g2[:, 1, :N].reshape(2 * N, HF)

    loss = pl.pallas_call(
        _fin_body,
        out_shape=jax.ShapeDtypeStruct((1, 1), jnp.float32),
    )(g2_lo, g2_hi, dd2, W2, b2.reshape(1, F), Wd)
    return loss[0, 0]


# speedup vs baseline: 1.0590x; 1.0590x over previous
import functools

import jax
import jax.numpy as jnp
from jax import lax
from jax.experimental import pallas as pl
from jax.experimental.pallas import tpu as pltpu
from jax.experimental.pallas import tpu_sc as plsc

N = 10000
E = 320000
F = 128
NS = 16
NC = 2
BLK = 128
K = 16
G = 10
EPT = G * K * BLK
EPAD = NS * EPT
NPAD = 10240
DT = NPAD // NS
RPT = NPAD // NS
GB = 128
NXB = NPAD // GB

_mesh = plsc.VectorSubcoreMesh(core_axis_name="c", subcore_axis_name="s")


@functools.partial(
    pl.kernel,
    out_type=(
        jax.ShapeDtypeStruct((NC, NPAD), jnp.float32),
        jax.ShapeDtypeStruct((NPAD, F), jnp.float32),
    ),
    mesh=_mesh,
    scratch_types=[
        pltpu.VMEM_SHARED((NPAD,), jnp.float32),
        pltpu.VMEM((2, K, BLK), jnp.int32),
        pltpu.VMEM((BLK,), jnp.float32),
        pltpu.VMEM((GB,), jnp.int32),
        pltpu.VMEM((GB, F), jnp.float32),
        pltpu.SemaphoreType.DMA,
        pltpu.SemaphoreType.DMA,
    ],
)
def _prep(sd_hbm, perm_hbm, x_hbm, ones_hbm, zeros_hbm,
          degs_hbm, xg_hbm,
          acc, idx_v, ones_v, perm_v, rows_v, hsem, gsem):
    c = lax.axis_index("c")
    s = lax.axis_index("s")
    wid = c * NS + s
    pltpu.sync_copy(zeros_hbm, acc.at[pl.ds(s * DT, DT)])
    pltpu.sync_copy(ones_hbm, ones_v)
    pltpu.sync_copy(sd_hbm.at[c, s, 0], idx_v.at[0])
    plsc.subcore_barrier()

    def hist_group(g, carry):
        cur = lax.rem(g, 2)
        nxt = 1 - cur
        for j in range(K):
            pltpu.async_copy(ones_v, acc.at[idx_v.at[cur, j]], hsem, add=True)

        @pl.when(g + 1 < G)
        def _stage():
            pltpu.sync_copy(sd_hbm.at[c, s, g + 1], idx_v.at[nxt])

        for j in range(K):
            pltpu.make_async_copy(ones_v, acc.at[idx_v.at[cur, j]], hsem).wait()
        return carry

    lax.fori_loop(0, G, hist_group, 0)

    for k in range(NXB // (NC * NS) + 1):
        b = wid + NC * NS * k

        @pl.when(b < NXB)
        def _xg_block(b=b):
            pltpu.sync_copy(perm_hbm.at[b], perm_v)
            pltpu.async_copy(x_hbm.at[perm_v], rows_v, gsem).wait()
            pltpu.sync_copy(rows_v, xg_hbm.at[pl.ds(b * GB, GB)])

    plsc.subcore_barrier()
    pltpu.sync_copy(acc.at[pl.ds(s * DT, DT)], degs_hbm.at[c, pl.ds(s * DT, DT)])


HF = F // 2
NBUF = 4
LC = 632
TAIL = N - (NS - 1) * LC


@functools.partial(
    pl.kernel,
    out_type=jax.ShapeDtypeStruct((NC, 2, NPAD, HF), jnp.float32),
    mesh=_mesh,
    scratch_types=[
        pltpu.VMEM_SHARED((N, HF), jnp.float32),
        pltpu.VMEM_SHARED((NPAD, HF), jnp.float32),
        pltpu.VMEM((2, K, BLK), jnp.int32),
        pltpu.VMEM((2, K, BLK), jnp.int32),
        pltpu.VMEM((NBUF, BLK, HF), jnp.float32),
        pltpu.SemaphoreType.DMA,
        pltpu.SemaphoreType.DMA,
        pltpu.SemaphoreType.DMA,
        pltpu.SemaphoreType.DMA,
    ],
    compiler_params=pltpu.CompilerParams(use_tc_tiling_on_sc=False),
)
def _spmm(tlo_hbm, thi_hbm, gidx_hbm, dst_hbm, zrows_hbm,
          out_hbm,
          tbl, acc, gi_v, di_v, rows_v, sem0, sem1, sem2, sem3):
    sems = (sem0, sem1, sem2, sem3)
    c = lax.axis_index("c")
    s = lax.axis_index("s")

    for p, t_hbm in enumerate((tlo_hbm, thi_hbm)):
        @pl.when(s < NS - 1)
        def _load_main():
            pltpu.sync_copy(t_hbm.at[pl.ds(c * N + s * LC, LC)],
                            tbl.at[pl.ds(s * LC, LC)])

        @pl.when(s == NS - 1)
        def _load_tail():
            pltpu.sync_copy(t_hbm.at[pl.ds(c * N + (NS - 1) * LC, TAIL)],
                            tbl.at[pl.ds((NS - 1) * LC, TAIL)])

        pltpu.sync_copy(zrows_hbm, acc.at[pl.ds(s * RPT, RPT)])
        pltpu.sync_copy(gidx_hbm.at[s, 0], gi_v.at[0])
        pltpu.sync_copy(dst_hbm.at[s, 0], di_v.at[0])
        plsc.subcore_barrier()

        for j0 in range(NBUF - 1):
            pltpu.async_copy(tbl.at[gi_v.at[0, j0]], rows_v.at[j0], sems[j0])

        def group(g, carry):
            cur = lax.rem(g, 2)
            nxt = 1 - cur

            @pl.when(g + 1 < G)
            def _stage_next():
                pltpu.sync_copy(gidx_hbm.at[s, g + 1], gi_v.at[nxt])
                pltpu.sync_copy(dst_hbm.at[s, g + 1], di_v.at[nxt])

            for j in range(K):
                b = j % NBUF
                ja = j + NBUF - 1
                ba = ja % NBUF
                if ja < K:
                    pltpu.async_copy(tbl.at[gi_v.at[cur, ja]],
                                     rows_v.at[ba], sems[ba])
                else:
                    @pl.when(g + 1 < G)
                    def _fire_cross():
                        pltpu.async_copy(tbl.at[gi_v.at[nxt, ja - K]],
                                         rows_v.at[ba], sems[ba])
                pltpu.make_async_copy(
                    tbl.at[gi_v.at[cur, j]], rows_v.at[b], sems[b]).wait()
                pltpu.sync_copy(rows_v.at[b], acc.at[di_v.at[cur, j]], add=True)
            return carry

        lax.fori_loop(0, G, group, 0)
        plsc.subcore_barrier()
        pltpu.sync_copy(acc.at[pl.ds(s * RPT, RPT)],
                        out_hbm.at[c, p, pl.ds(s * RPT, RPT)])
        if p == 0:
            plsc.subcore_barrier()


def _norm(d):
    return jnp.where(d > 0.5, lax.rsqrt(d), 0.0)


def _t1_body(x_ref, xg_ref, ds_ref, lo_ref, hi_ref):
    ns = _norm(ds_ref[...])
    lo_ref[0:N] = x_ref[:, 0:HF] * ns
    lo_ref[N:2 * N] = xg_ref[0:N, 0:HF] * ns
    hi_ref[0:N] = x_ref[:, HF:F] * ns
    hi_ref[N:2 * N] = xg_ref[0:N, HF:F] * ns


def _mm_body(glo_ref, ghi_ref, dd_ref, ds_ref, w_ref, b_ref, lo_ref, hi_ref):
    g = jnp.concatenate([glo_ref[...], ghi_ref[...]], axis=1)
    g = g * _norm(dd_ref[...])
    h = jnp.dot(g, w_ref[...], preferred_element_type=jnp.float32) + b_ref[...]
    h = jnp.maximum(h, 0.0) * _norm(ds_ref[...])
    lo_ref[...] = h[:, 0:HF]
    hi_ref[...] = h[:, HF:F]


def _fin_body(glo_ref, ghi_ref, dd_ref, w2_ref, b2_ref, wd_ref, o_ref):
    gs = jnp.concatenate([glo_ref[...], ghi_ref[...]], axis=1)
    gs = gs * _norm(dd_ref[...])
    gp = gs[0:N]
    gn = gs[N:2 * N]
    u = jnp.sum(gp, axis=0, keepdims=True) * (1.0 / N)
    sm = jnp.dot(u, w2_ref[...], preferred_element_type=jnp.float32) + b2_ref[...]
    tt = (((1,), (1,)), ((), ()))
    vv = lax.dot_general(sm, wd_ref[...], tt,
                         preferred_element_type=jnp.float32)
    w2v = lax.dot_general(vv, w2_ref[...], tt,
                          preferred_element_type=jnp.float32)
    cc = jnp.sum(b2_ref[...] * vv, axis=1, keepdims=True)
    logits = jnp.sum(gs * w2v, axis=1, keepdims=True) + cc
    lp = logits[0:N]
    ln = logits[N:2 * N]

    def softplus(z):
        return jnp.maximum(z, 0.0) + jnp.log1p(jnp.exp(-jnp.abs(z)))

    l1 = jnp.sum(softplus(-lp), axis=0, keepdims=True) * (1.0 / N)
    l2 = jnp.sum(softplus(ln), axis=0, keepdims=True) * (1.0 / N)
    o_ref[...] = l1 + l2


def kernel(features, edge_index, perm, W1, b1, W2, b2, Wd):
    src = edge_index[0]
    dst = edge_index[1]
    npad = EPAD - E
    junk = (N + jax.lax.iota(jnp.int32, npad) % (NPAD - N))
    src_h = jnp.concatenate([src, junk]).reshape(NS, G, K, BLK)
    dst_p = jnp.concatenate([dst, junk]).reshape(NS, G, K, BLK)
    sd = jnp.stack([src_h, dst_p])
    gidx = jnp.concatenate(
        [src, jnp.zeros((npad,), jnp.int32)]).reshape(NS, G, K, BLK)
    perm_pad = jnp.concatenate(
        [perm, jnp.zeros((NPAD - N,), jnp.int32)]).reshape(NXB, GB)
    ones = jnp.ones((BLK,), jnp.float32)
    zeros = jnp.zeros((DT,), jnp.float32)
    zrows = jnp.zeros((RPT, HF), jnp.float32)

    degs, xg = _prep(sd, perm_pad, features, ones, zeros)
    ds_col = degs[0, :N].reshape(N, 1)
    dd_col = degs[1, :N].reshape(N, 1)
    dd2 = jnp.concatenate([dd_col, dd_col], axis=0)
    ds2 = jnp.concatenate([ds_col, ds_col], axis=0)

    half = jax.ShapeDtypeStruct((2 * N, HF), jnp.float32)
    t1_lo, t1_hi = pl.pallas_call(
        _t1_body, out_shape=(half, half))(features, xg, ds_col)

    g1 = _spmm(t1_lo, t1_hi, gidx, dst_p, zrows)
    g1_lo = g1[:, 0, :N].reshape(2 * N, HF)
    g1_hi = g1[:, 1, :N].reshape(2 * N, HF)

    RB = 2000
    t2_lo, t2_hi = pl.pallas_call(
        _mm_body,
        grid=(2 * N // RB,),
        in_specs=[
            pl.BlockSpec((RB, HF), lambda i: (i, 0)),
            pl.BlockSpec((RB, HF), lambda i: (i, 0)),
            pl.BlockSpec((RB, 1), lambda i: (i, 0)),
            pl.BlockSpec((RB, 1), lambda i: (i, 0)),
            pl.BlockSpec((F, F), lambda i: (0, 0)),
            pl.BlockSpec((1, F), lambda i: (0, 0)),
        ],
        out_specs=[
            pl.BlockSpec((RB, HF), lambda i: (i, 0)),
            pl.BlockSpec((RB, HF), lambda i: (i, 0)),
        ],
        out_shape=(half, half))(
            g1_lo, g1_hi, dd2, ds2, W1, b1.reshape(1, F))

    g2 = _spmm(t2_lo, t2_hi, gidx, dst_p, zrows)
    g2_lo = g2[:, 0, :N].reshape(2 * N, HF)
    g2_hi = g2[:, 1, :N].reshape(2 * N, HF)

    loss = pl.pallas_call(
        _fin_body,
        out_shape=jax.ShapeDtypeStruct((1, 1), jnp.float32),
    )(g2_lo, g2_hi, dd2, W2, b2.reshape(1, F), Wd)
    return loss[0, 0]


# speedup vs baseline: 1.0647x; 1.0054x over previous
import functools

import jax
import jax.numpy as jnp
from jax import lax
from jax.experimental import pallas as pl
from jax.experimental.pallas import tpu as pltpu
from jax.experimental.pallas import tpu_sc as plsc

N = 10000
E = 320000
F = 128
NS = 16
NC = 2
BLK = 128
K = 20
G = 8
EPT = G * K * BLK
EPAD = NS * EPT
NPAD = 10240
DT = NPAD // NS
RPT = NPAD // NS
GB = 128
NXB = NPAD // GB

_mesh = plsc.VectorSubcoreMesh(core_axis_name="c", subcore_axis_name="s")


@functools.partial(
    pl.kernel,
    out_type=(
        jax.ShapeDtypeStruct((NC, NPAD), jnp.float32),
        jax.ShapeDtypeStruct((NPAD, F), jnp.float32),
    ),
    mesh=_mesh,
    scratch_types=[
        pltpu.VMEM_SHARED((NPAD,), jnp.float32),
        pltpu.VMEM((2, K, BLK), jnp.int32),
        pltpu.VMEM((BLK,), jnp.float32),
        pltpu.VMEM((GB,), jnp.int32),
        pltpu.VMEM((GB, F), jnp.float32),
        pltpu.SemaphoreType.DMA,
        pltpu.SemaphoreType.DMA,
    ],
)
def _prep(sd_hbm, perm_hbm, x_hbm, ones_hbm, zeros_hbm,
          degs_hbm, xg_hbm,
          acc, idx_v, ones_v, perm_v, rows_v, hsem, gsem):
    c = lax.axis_index("c")
    s = lax.axis_index("s")
    wid = c * NS + s
    pltpu.sync_copy(zeros_hbm, acc.at[pl.ds(s * DT, DT)])
    pltpu.sync_copy(ones_hbm, ones_v)
    pltpu.sync_copy(sd_hbm.at[c, s, 0], idx_v.at[0])
    plsc.subcore_barrier()

    def hist_group(g, carry):
        cur = lax.rem(g, 2)
        nxt = 1 - cur
        for j in range(K):
            pltpu.async_copy(ones_v, acc.at[idx_v.at[cur, j]], hsem, add=True)

        @pl.when(g + 1 < G)
        def _stage():
            pltpu.sync_copy(sd_hbm.at[c, s, g + 1], idx_v.at[nxt])

        for j in range(K):
            pltpu.make_async_copy(ones_v, acc.at[idx_v.at[cur, j]], hsem).wait()
        return carry

    lax.fori_loop(0, G, hist_group, 0)

    for k in range(NXB // (NC * NS) + 1):
        b = wid + NC * NS * k

        @pl.when(b < NXB)
        def _xg_block(b=b):
            pltpu.sync_copy(perm_hbm.at[b], perm_v)
            pltpu.async_copy(x_hbm.at[perm_v], rows_v, gsem).wait()
            pltpu.sync_copy(rows_v, xg_hbm.at[pl.ds(b * GB, GB)])

    plsc.subcore_barrier()
    pltpu.sync_copy(acc.at[pl.ds(s * DT, DT)], degs_hbm.at[c, pl.ds(s * DT, DT)])


HF = F // 2
NBUF = 4
LC = 632
TAIL = N - (NS - 1) * LC


@functools.partial(
    pl.kernel,
    out_type=jax.ShapeDtypeStruct((NC, 2, NPAD, HF), jnp.float32),
    mesh=_mesh,
    scratch_types=[
        pltpu.VMEM_SHARED((N, HF), jnp.float32),
        pltpu.VMEM_SHARED((NPAD, HF), jnp.float32),
        pltpu.VMEM((2, K, BLK), jnp.int32),
        pltpu.VMEM((2, K, BLK), jnp.int32),
        pltpu.VMEM((NBUF, BLK, HF), jnp.float32),
        pltpu.SemaphoreType.DMA,
        pltpu.SemaphoreType.DMA,
        pltpu.SemaphoreType.DMA,
        pltpu.SemaphoreType.DMA,
    ],
    compiler_params=pltpu.CompilerParams(use_tc_tiling_on_sc=False),
)
def _spmm(tlo_hbm, thi_hbm, gidx_hbm, dst_hbm, zrows_hbm,
          out_hbm,
          tbl, acc, gi_v, di_v, rows_v, sem0, sem1, sem2, sem3):
    sems = (sem0, sem1, sem2, sem3)
    c = lax.axis_index("c")
    s = lax.axis_index("s")

    for p, t_hbm in enumerate((tlo_hbm, thi_hbm)):
        @pl.when(s < NS - 1)
        def _load_main():
            pltpu.sync_copy(t_hbm.at[pl.ds(c * N + s * LC, LC)],
                            tbl.at[pl.ds(s * LC, LC)])

        @pl.when(s == NS - 1)
        def _load_tail():
            pltpu.sync_copy(t_hbm.at[pl.ds(c * N + (NS - 1) * LC, TAIL)],
                            tbl.at[pl.ds((NS - 1) * LC, TAIL)])

        pltpu.sync_copy(zrows_hbm, acc.at[pl.ds(s * RPT, RPT)])
        pltpu.sync_copy(gidx_hbm.at[s, 0], gi_v.at[0])
        pltpu.sync_copy(dst_hbm.at[s, 0], di_v.at[0])
        plsc.subcore_barrier()

        for j0 in range(NBUF - 1):
            pltpu.async_copy(tbl.at[gi_v.at[0, j0]], rows_v.at[j0], sems[j0])

        def group(g, carry):
            cur = lax.rem(g, 2)
            nxt = 1 - cur

            @pl.when(g + 1 < G)
            def _stage_next():
                pltpu.sync_copy(gidx_hbm.at[s, g + 1], gi_v.at[nxt])
                pltpu.sync_copy(dst_hbm.at[s, g + 1], di_v.at[nxt])

            for j in range(K):
                b = j % NBUF
                ja = j + NBUF - 1
                ba = ja % NBUF
                if ja < K:
                    pltpu.async_copy(tbl.at[gi_v.at[cur, ja]],
                                     rows_v.at[ba], sems[ba])
                else:
                    @pl.when(g + 1 < G)
                    def _fire_cross():
                        pltpu.async_copy(tbl.at[gi_v.at[nxt, ja - K]],
                                         rows_v.at[ba], sems[ba])
                pltpu.make_async_copy(
                    tbl.at[gi_v.at[cur, j]], rows_v.at[b], sems[b]).wait()
                pltpu.sync_copy(rows_v.at[b], acc.at[di_v.at[cur, j]], add=True)
            return carry

        lax.fori_loop(0, G, group, 0)
        plsc.subcore_barrier()
        pltpu.sync_copy(acc.at[pl.ds(s * RPT, RPT)],
                        out_hbm.at[c, p, pl.ds(s * RPT, RPT)])
        if p == 0:
            plsc.subcore_barrier()


def _norm(d):
    return jnp.where(d > 0.5, lax.rsqrt(d), 0.0)


def _t1_body(x_ref, xg_ref, ds_ref, lo_ref, hi_ref):
    ns = _norm(ds_ref[...])
    lo_ref[0:N] = x_ref[:, 0:HF] * ns
    lo_ref[N:2 * N] = xg_ref[0:N, 0:HF] * ns
    hi_ref[0:N] = x_ref[:, HF:F] * ns
    hi_ref[N:2 * N] = xg_ref[0:N, HF:F] * ns


def _mm_body(glo_ref, ghi_ref, dd_ref, ds_ref, w_ref, b_ref, lo_ref, hi_ref):
    g = jnp.concatenate([glo_ref[...], ghi_ref[...]], axis=1)
    g = g * _norm(dd_ref[...])
    h = jnp.dot(g, w_ref[...], preferred_element_type=jnp.float32) + b_ref[...]
    h = jnp.maximum(h, 0.0) * _norm(ds_ref[...])
    lo_ref[...] = h[:, 0:HF]
    hi_ref[...] = h[:, HF:F]


def _fin_body(glo_ref, ghi_ref, dd_ref, w2_ref, b2_ref, wd_ref, o_ref):
    gs = jnp.concatenate([glo_ref[...], ghi_ref[...]], axis=1)
    gs = gs * _norm(dd_ref[...])
    gp = gs[0:N]
    gn = gs[N:2 * N]
    u = jnp.sum(gp, axis=0, keepdims=True) * (1.0 / N)
    sm = jnp.dot(u, w2_ref[...], preferred_element_type=jnp.float32) + b2_ref[...]
    tt = (((1,), (1,)), ((), ()))
    vv = lax.dot_general(sm, wd_ref[...], tt,
                         preferred_element_type=jnp.float32)
    w2v = lax.dot_general(vv, w2_ref[...], tt,
                          preferred_element_type=jnp.float32)
    cc = jnp.sum(b2_ref[...] * vv, axis=1, keepdims=True)
    logits = jnp.sum(gs * w2v, axis=1, keepdims=True) + cc
    lp = logits[0:N]
    ln = logits[N:2 * N]

    def softplus(z):
        return jnp.maximum(z, 0.0) + jnp.log1p(jnp.exp(-jnp.abs(z)))

    l1 = jnp.sum(softplus(-lp), axis=0, keepdims=True) * (1.0 / N)
    l2 = jnp.sum(softplus(ln), axis=0, keepdims=True) * (1.0 / N)
    o_ref[...] = l1 + l2


def kernel(features, edge_index, perm, W1, b1, W2, b2, Wd):
    src = edge_index[0]
    dst = edge_index[1]
    npad = EPAD - E
    junk = (N + jax.lax.iota(jnp.int32, npad) % (NPAD - N))
    src_h = jnp.concatenate([src, junk]).reshape(NS, G, K, BLK)
    dst_p = jnp.concatenate([dst, junk]).reshape(NS, G, K, BLK)
    sd = jnp.stack([src_h, dst_p])
    gidx = jnp.concatenate(
        [src, jnp.zeros((npad,), jnp.int32)]).reshape(NS, G, K, BLK)
    perm_pad = jnp.concatenate(
        [perm, jnp.zeros((NPAD - N,), jnp.int32)]).reshape(NXB, GB)
    ones = jnp.ones((BLK,), jnp.float32)
    zeros = jnp.zeros((DT,), jnp.float32)
    zrows = jnp.zeros((RPT, HF), jnp.float32)

    degs, xg = _prep(sd, perm_pad, features, ones, zeros)
    ds_col = degs[0, :N].reshape(N, 1)
    dd_col = degs[1, :N].reshape(N, 1)
    dd2 = jnp.concatenate([dd_col, dd_col], axis=0)
    ds2 = jnp.concatenate([ds_col, ds_col], axis=0)

    half = jax.ShapeDtypeStruct((2 * N, HF), jnp.float32)
    t1_lo, t1_hi = pl.pallas_call(
        _t1_body, out_shape=(half, half))(features, xg, ds_col)

    g1 = _spmm(t1_lo, t1_hi, gidx, dst_p, zrows)
    g1_lo = g1[:, 0, :N].reshape(2 * N, HF)
    g1_hi = g1[:, 1, :N].reshape(2 * N, HF)

    RB = 2000
    t2_lo, t2_hi = pl.pallas_call(
        _mm_body,
        grid=(2 * N // RB,),
        in_specs=[
            pl.BlockSpec((RB, HF), lambda i: (i, 0)),
            pl.BlockSpec((RB, HF), lambda i: (i, 0)),
            pl.BlockSpec((RB, 1), lambda i: (i, 0)),
            pl.BlockSpec((RB, 1), lambda i: (i, 0)),
            pl.BlockSpec((F, F), lambda i: (0, 0)),
            pl.BlockSpec((1, F), lambda i: (0, 0)),
        ],
        out_specs=[
            pl.BlockSpec((RB, HF), lambda i: (i, 0)),
            pl.BlockSpec((RB, HF), lambda i: (i, 0)),
        ],
        out_shape=(half, half))(
            g1_lo, g1_hi, dd2, ds2, W1, b1.reshape(1, F))

    g2 = _spmm(t2_lo, t2_hi, gidx, dst_p, zrows)
    g2_lo = g2[:, 0, :N].reshape(2 * N, HF)
    g2_hi = g2[:, 1, :N].reshape(2 * N, HF)

    loss = pl.pallas_call(
        _fin_body,
        out_shape=jax.ShapeDtypeStruct((1, 1), jnp.float32),
    )(g2_lo, g2_hi, dd2, W2, b2.reshape(1, F), Wd)
    return loss[0, 0]
